# baseline (speedup 1.0000x reference)
import functools

import jax
import jax.numpy as jnp
from jax.experimental import pallas as pl
from jax.experimental.pallas import tpu as pltpu

_BM = 1024
_CH = 2048
_SCH = 2048
_TOL = 1e-4
_EPS = 1e-8


def _router_kernel(x_ref, w_ref, scores_ref, idx_ref,
                   cost_ref, acc_ref, d1_ref,
                   *, M, E, NB, BM, NCH, CH, tol):
    i = pl.program_id(0)

    @pl.when(i < NB)
    def _matmul_step():
        logits = jax.lax.dot_general(
            x_ref[...], w_ref[...],
            dimension_numbers=(((1,), (1,)), ((), ())),
            preferred_element_type=jnp.float32,
        )
        costb = jnp.exp(logits)
        cost_ref[pl.ds(i * BM, BM), :] = costb
        r = jnp.sum(costb, axis=1, keepdims=True)
        d0b = (1.0 / M) / (r + _EPS)
        part = jnp.sum(costb * d0b, axis=0, keepdims=True)

        @pl.when(i == 0)
        def _():
            acc_ref[...] = part

        @pl.when(i > 0)
        def _():
            acc_ref[...] = acc_ref[...] + part

    @pl.when(i == NB - 1)
    def _sinkhorn_tail():
        ones = jnp.ones((1, E), jnp.float32)
        d1_1 = (1.0 / E) / (acc_ref[...] + _EPS)
        err1 = jnp.mean(jnp.abs(d1_1 - ones))
        d1_ref[...] = d1_1

        nch2 = M // _SCH

        def one_pass(d1):
            def body(c, acc):
                chunk = cost_ref[pl.ds(c * _SCH, _SCH), :]
                r = jnp.sum(chunk * d1, axis=1, keepdims=True)
                d0c = (1.0 / M) / (r + _EPS)
                return acc + jnp.sum(chunk * d0c, axis=0, keepdims=True)

            acc = jax.lax.fori_loop(0, nch2, body,
                                    jnp.zeros((1, E), jnp.float32))
            return (1.0 / E) / (acc + _EPS)

        def cond_fn(carry):
            err, it = carry
            return jnp.logical_and(err > tol, it < 200)

        def body_fn(carry):
            _, it = carry
            d1 = d1_ref[...]
            d1n = one_pass(d1)
            err = jnp.mean(jnp.abs(d1n - d1))
            d1_ref[...] = d1n
            return (err, it + 1)

        jax.lax.while_loop(cond_fn, body_fn,
                           (err1, jnp.asarray(1, jnp.int32)))

    @pl.when(i >= NB)
    def _topk_step():
        c = i - NB
        d1 = d1_ref[...]
        SUB = 256
        iota = jax.lax.broadcasted_iota(jnp.int32, (SUB, E), 1)

        def sub_step(s, _):
            chunk = cost_ref[pl.ds(c * CH + s * SUB, SUB), :]
            t = chunk * d1
            v1 = jnp.max(t, axis=1, keepdims=True)
            i1 = jnp.min(jnp.where(t == v1, iota, E), axis=1, keepdims=True)
            m1 = iota == i1
            t2 = jnp.where(m1, -jnp.inf, t)
            v2 = jnp.max(t2, axis=1, keepdims=True)
            i2 = jnp.min(jnp.where(t2 == v2, iota, E), axis=1, keepdims=True)
            rs = jnp.sum(chunk, axis=1, keepdims=True)
            c1 = jnp.sum(jnp.where(m1, chunk, 0.0), axis=1, keepdims=True)
            c2 = jnp.sum(jnp.where(iota == i2, chunk, 0.0), axis=1,
                         keepdims=True)
            scores_ref[pl.ds(s * SUB, SUB), :] = jnp.concatenate(
                [c1 / rs, c2 / rs], axis=1)
            idx_ref[pl.ds(s * SUB, SUB), :] = jnp.concatenate([i1, i2], axis=1)
            return 0

        jax.lax.fori_loop(0, CH // SUB, sub_step, 0)


def kernel(x, W):
    M = x.shape[0] * x.shape[1]
    K = x.shape[2]
    E = W.shape[0]
    xf = x.reshape(M, K)
    NB = M // _BM
    NCH = M // _CH

    scores, idx = pl.pallas_call(
        functools.partial(_router_kernel, M=M, E=E, NB=NB, BM=_BM,
                          NCH=NCH, CH=_CH, tol=_TOL),
        grid=(NB + NCH,),
        in_specs=[
            pl.BlockSpec((_BM, K), lambda i, nb=NB: (jnp.minimum(i, nb - 1), 0)),
            pl.BlockSpec((E, K), lambda i: (0, 0)),
        ],
        out_specs=[
            pl.BlockSpec((_CH, 2), lambda i, nb=NB: (jnp.maximum(i - nb, 0), 0)),
            pl.BlockSpec((_CH, 2), lambda i, nb=NB: (jnp.maximum(i - nb, 0), 0)),
        ],
        out_shape=[
            jax.ShapeDtypeStruct((M, 2), jnp.float32),
            jax.ShapeDtypeStruct((M, 2), jnp.int32),
        ],
        scratch_shapes=[
            pltpu.VMEM((M, E), jnp.float32),
            pltpu.VMEM((1, E), jnp.float32),
            pltpu.VMEM((1, E), jnp.float32),
        ],
    )(xf, W)
    return (scores, idx)


# speedup vs baseline: 1.0456x; 1.0456x over previous
import functools

import jax
import jax.numpy as jnp
from jax.experimental import pallas as pl
from jax.experimental.pallas import tpu as pltpu

_BM = 1024
_CH = 2048
_SCH = 2048
_TOL = 1e-4
_EPS = 1e-8


def _router_kernel(x_ref, w_ref, scores_ref, idx_ref,
                   cost_ref, acc_ref, d1_ref,
                   *, M, E, NB, BM, NCH, CH, tol):
    i = pl.program_id(0)

    @pl.when(i < NB)
    def _matmul_step():
        logits = jax.lax.dot_general(
            x_ref[...], w_ref[...],
            dimension_numbers=(((1,), (1,)), ((), ())),
            preferred_element_type=jnp.float32,
        )
        costb = jnp.exp(logits)
        cost_ref[pl.ds(i * BM, BM), :] = costb
        r = jnp.sum(costb, axis=1, keepdims=True)
        d0b = (1.0 / M) / (r + _EPS)
        part = jnp.sum(costb * d0b, axis=0, keepdims=True)

        @pl.when(i == 0)
        def _():
            acc_ref[...] = part

        @pl.when(i > 0)
        def _():
            acc_ref[...] = acc_ref[...] + part

    @pl.when(i == NB - 1)
    def _sinkhorn_tail():
        ones = jnp.ones((1, E), jnp.float32)
        d1_1 = (1.0 / E) / (acc_ref[...] + _EPS)
        err1 = jnp.mean(jnp.abs(d1_1 - ones))
        d1_ref[...] = d1_1

        nch2 = M // _SCH

        def one_pass(d1):
            def body(c, acc):
                chunk = cost_ref[pl.ds(c * _SCH, _SCH), :]
                r = jnp.sum(chunk * d1, axis=1, keepdims=True)
                d0c = (1.0 / M) / (r + _EPS)
                return acc + jnp.sum(chunk * d0c, axis=0, keepdims=True)

            acc = jax.lax.fori_loop(0, nch2, body,
                                    jnp.zeros((1, E), jnp.float32))
            return (1.0 / E) / (acc + _EPS)

        def cond_fn(carry):
            err, it = carry
            return jnp.logical_and(err > tol, it < 200)

        def body_fn(carry):
            _, it = carry
            d1 = d1_ref[...]
            d1n = one_pass(d1)
            err = jnp.mean(jnp.abs(d1n - d1))
            d1_ref[...] = d1n
            return (err, it + 1)

        jax.lax.while_loop(cond_fn, body_fn,
                           (err1, jnp.asarray(1, jnp.int32)))

    @pl.when(i >= NB)
    def _topk_step():
        c = i - NB
        chunk = cost_ref[pl.ds(c * CH, CH), :]
        d1 = d1_ref[...]
        t = chunk * d1
        iota = jax.lax.broadcasted_iota(jnp.int32, (CH, E), 1)
        v1 = jnp.max(t, axis=1, keepdims=True)
        i1 = jnp.min(jnp.where(t == v1, iota, E), axis=1, keepdims=True)
        m1 = iota == i1
        t2 = jnp.where(m1, -jnp.inf, t)
        v2 = jnp.max(t2, axis=1, keepdims=True)
        i2 = jnp.min(jnp.where(t2 == v2, iota, E), axis=1, keepdims=True)
        rs = jnp.sum(chunk, axis=1, keepdims=True)
        c1 = jnp.sum(jnp.where(m1, chunk, 0.0), axis=1, keepdims=True)
        c2 = jnp.sum(jnp.where(iota == i2, chunk, 0.0), axis=1, keepdims=True)
        scores_ref[...] = jnp.concatenate([c1 / rs, c2 / rs], axis=1)
        idx_ref[...] = jnp.concatenate([i1, i2], axis=1)


def kernel(x, W):
    M = x.shape[0] * x.shape[1]
    K = x.shape[2]
    E = W.shape[0]
    xf = x.reshape(M, K)
    NB = M // _BM
    NCH = M // _CH

    scores, idx = pl.pallas_call(
        functools.partial(_router_kernel, M=M, E=E, NB=NB, BM=_BM,
                          NCH=NCH, CH=_CH, tol=_TOL),
        grid=(NB + NCH,),
        in_specs=[
            pl.BlockSpec((_BM, K), lambda i, nb=NB: (jnp.minimum(i, nb - 1), 0)),
            pl.BlockSpec((E, K), lambda i: (0, 0)),
        ],
        out_specs=[
            pl.BlockSpec((_CH, 2), lambda i, nb=NB: (jnp.maximum(i - nb, 0), 0)),
            pl.BlockSpec((_CH, 2), lambda i, nb=NB: (jnp.maximum(i - nb, 0), 0)),
        ],
        out_shape=[
            jax.ShapeDtypeStruct((M, 2), jnp.float32),
            jax.ShapeDtypeStruct((M, 2), jnp.int32),
        ],
        scratch_shapes=[
            pltpu.VMEM((M, E), jnp.float32),
            pltpu.VMEM((1, E), jnp.float32),
            pltpu.VMEM((1, E), jnp.float32),
        ],
    )(xf, W)
    return (scores, idx)


# speedup vs baseline: 1.0475x; 1.0017x over previous
import functools

import jax
import jax.numpy as jnp
from jax.experimental import pallas as pl
from jax.experimental.pallas import tpu as pltpu

_BM = 512
_CH = 4096
_SCH = 2048
_TOL = 1e-4
_EPS = 1e-8


def _router_kernel(x_ref, w_ref, scores_ref, idx_ref,
                   cost_ref, acc_ref, d1_ref,
                   *, M, E, NB, BM, NCH, CH, tol):
    i = pl.program_id(0)

    @pl.when(i < NB)
    def _matmul_step():
        logits = jax.lax.dot_general(
            x_ref[...], w_ref[...],
            dimension_numbers=(((1,), (1,)), ((), ())),
            preferred_element_type=jnp.float32,
        )
        costb = jnp.exp(logits)
        cost_ref[pl.ds(i * BM, BM), :] = costb
        r = jnp.sum(costb, axis=1, keepdims=True)
        d0b = (1.0 / M) / (r + _EPS)
        part = jnp.sum(costb * d0b, axis=0, keepdims=True)

        @pl.when(i == 0)
        def _():
            acc_ref[...] = part

        @pl.when(i > 0)
        def _():
            acc_ref[...] = acc_ref[...] + part

    @pl.when(i == NB - 1)
    def _sinkhorn_tail():
        ones = jnp.ones((1, E), jnp.float32)
        d1_1 = (1.0 / E) / (acc_ref[...] + _EPS)
        err1 = jnp.mean(jnp.abs(d1_1 - ones))
        d1_ref[...] = d1_1

        nch2 = M // _SCH

        def one_pass(d1):
            def body(c, acc):
                chunk = cost_ref[pl.ds(c * _SCH, _SCH), :]
                r = jnp.sum(chunk * d1, axis=1, keepdims=True)
                d0c = (1.0 / M) / (r + _EPS)
                return acc + jnp.sum(chunk * d0c, axis=0, keepdims=True)

            acc = jax.lax.fori_loop(0, nch2, body,
                                    jnp.zeros((1, E), jnp.float32))
            return (1.0 / E) / (acc + _EPS)

        def cond_fn(carry):
            err, it = carry
            return jnp.logical_and(err > tol, it < 200)

        def body_fn(carry):
            _, it = carry
            d1 = d1_ref[...]
            d1n = one_pass(d1)
            err = jnp.mean(jnp.abs(d1n - d1))
            d1_ref[...] = d1n
            return (err, it + 1)

        jax.lax.while_loop(cond_fn, body_fn,
                           (err1, jnp.asarray(1, jnp.int32)))

    @pl.when(i >= NB)
    def _topk_step():
        c = i - NB
        chunk = cost_ref[pl.ds(c * CH, CH), :]
        d1 = d1_ref[...]
        t = chunk * d1
        iota = jax.lax.broadcasted_iota(jnp.int32, (CH, E), 1)
        v1 = jnp.max(t, axis=1, keepdims=True)
        i1 = jnp.min(jnp.where(t == v1, iota, E), axis=1, keepdims=True)
        m1 = iota == i1
        t2 = jnp.where(m1, -jnp.inf, t)
        v2 = jnp.max(t2, axis=1, keepdims=True)
        i2 = jnp.min(jnp.where(t2 == v2, iota, E), axis=1, keepdims=True)
        rs = jnp.sum(chunk, axis=1, keepdims=True)
        c1 = jnp.sum(jnp.where(m1, chunk, 0.0), axis=1, keepdims=True)
        c2 = jnp.sum(jnp.where(iota == i2, chunk, 0.0), axis=1, keepdims=True)
        scores_ref[...] = jnp.concatenate([c1 / rs, c2 / rs], axis=1)
        idx_ref[...] = jnp.concatenate([i1, i2], axis=1)


def kernel(x, W):
    M = x.shape[0] * x.shape[1]
    K = x.shape[2]
    E = W.shape[0]
    xf = x.reshape(M, K)
    NB = M // _BM
    NCH = M // _CH

    scores, idx = pl.pallas_call(
        functools.partial(_router_kernel, M=M, E=E, NB=NB, BM=_BM,
                          NCH=NCH, CH=_CH, tol=_TOL),
        grid=(NB + NCH,),
        in_specs=[
            pl.BlockSpec((_BM, K), lambda i, nb=NB: (jnp.minimum(i, nb - 1), 0)),
            pl.BlockSpec((E, K), lambda i: (0, 0)),
        ],
        out_specs=[
            pl.BlockSpec((_CH, 2), lambda i, nb=NB: (jnp.maximum(i - nb, 0), 0)),
            pl.BlockSpec((_CH, 2), lambda i, nb=NB: (jnp.maximum(i - nb, 0), 0)),
        ],
        out_shape=[
            jax.ShapeDtypeStruct((M, 2), jnp.float32),
            jax.ShapeDtypeStruct((M, 2), jnp.int32),
        ],
        scratch_shapes=[
            pltpu.VMEM((M, E), jnp.float32),
            pltpu.VMEM((1, E), jnp.float32),
            pltpu.VMEM((1, E), jnp.float32),
        ],
    )(xf, W)
    return (scores, idx)


# speedup vs baseline: 1.0485x; 1.0010x over previous
import functools

import jax
import jax.numpy as jnp
from jax.experimental import pallas as pl
from jax.experimental.pallas import tpu as pltpu

_BM = 512
_CH = 4096
_SCH = 4096
_TOL = 1e-4
_EPS = 1e-8


def _router_kernel(x_ref, w_ref, scores_ref, idx_ref,
                   cost_ref, acc_ref, d1_ref,
                   *, M, E, NB, BM, NCH, CH, tol):
    i = pl.program_id(0)

    @pl.when(i < NB)
    def _matmul_step():
        logits = jax.lax.dot_general(
            x_ref[...], w_ref[...],
            dimension_numbers=(((1,), (1,)), ((), ())),
            preferred_element_type=jnp.float32,
        )
        costb = jnp.exp(logits)
        cost_ref[pl.ds(i * BM, BM), :] = costb
        r = jnp.sum(costb, axis=1, keepdims=True)
        d0b = (1.0 / M) / (r + _EPS)
        part = jnp.sum(costb * d0b, axis=0, keepdims=True)

        @pl.when(i == 0)
        def _():
            acc_ref[...] = part

        @pl.when(i > 0)
        def _():
            acc_ref[...] = acc_ref[...] + part

    @pl.when(i == NB - 1)
    def _sinkhorn_tail():
        ones = jnp.ones((1, E), jnp.float32)
        d1_1 = (1.0 / E) / (acc_ref[...] + _EPS)
        err1 = jnp.mean(jnp.abs(d1_1 - ones))
        d1_ref[...] = d1_1

        nch2 = M // _SCH

        def one_pass(d1):
            def body(c, acc):
                chunk = cost_ref[pl.ds(c * _SCH, _SCH), :]
                r = jnp.sum(chunk * d1, axis=1, keepdims=True)
                d0c = (1.0 / M) / (r + _EPS)
                return acc + jnp.sum(chunk * d0c, axis=0, keepdims=True)

            acc = jax.lax.fori_loop(0, nch2, body,
                                    jnp.zeros((1, E), jnp.float32))
            return (1.0 / E) / (acc + _EPS)

        def cond_fn(carry):
            err, it = carry
            return jnp.logical_and(err > tol, it < 200)

        def body_fn(carry):
            _, it = carry
            d1 = d1_ref[...]
            d1n = one_pass(d1)
            err = jnp.mean(jnp.abs(d1n - d1))
            d1_ref[...] = d1n
            return (err, it + 1)

        jax.lax.while_loop(cond_fn, body_fn,
                           (err1, jnp.asarray(1, jnp.int32)))

    @pl.when(i >= NB)
    def _topk_step():
        c = i - NB
        chunk = cost_ref[pl.ds(c * CH, CH), :]
        d1 = d1_ref[...]
        t = chunk * d1
        iota = jax.lax.broadcasted_iota(jnp.int32, (CH, E), 1)
        v1 = jnp.max(t, axis=1, keepdims=True)
        i1 = jnp.min(jnp.where(t == v1, iota, E), axis=1, keepdims=True)
        m1 = iota == i1
        t2 = jnp.where(m1, -jnp.inf, t)
        v2 = jnp.max(t2, axis=1, keepdims=True)
        i2 = jnp.min(jnp.where(t2 == v2, iota, E), axis=1, keepdims=True)
        rs = jnp.sum(chunk, axis=1, keepdims=True)
        c1 = jnp.sum(jnp.where(m1, chunk, 0.0), axis=1, keepdims=True)
        c2 = jnp.sum(jnp.where(iota == i2, chunk, 0.0), axis=1, keepdims=True)
        scores_ref[...] = jnp.concatenate([c1 / rs, c2 / rs], axis=1)
        idx_ref[...] = jnp.concatenate([i1, i2], axis=1)


def kernel(x, W):
    M = x.shape[0] * x.shape[1]
    K = x.shape[2]
    E = W.shape[0]
    xf = x.reshape(M, K)
    NB = M // _BM
    NCH = M // _CH

    scores, idx = pl.pallas_call(
        functools.partial(_router_kernel, M=M, E=E, NB=NB, BM=_BM,
                          NCH=NCH, CH=_CH, tol=_TOL),
        grid=(NB + NCH,),
        in_specs=[
            pl.BlockSpec((_BM, K), lambda i, nb=NB: (jnp.minimum(i, nb - 1), 0)),
            pl.BlockSpec((E, K), lambda i: (0, 0)),
        ],
        out_specs=[
            pl.BlockSpec((_CH, 2), lambda i, nb=NB: (jnp.maximum(i - nb, 0), 0)),
            pl.BlockSpec((_CH, 2), lambda i, nb=NB: (jnp.maximum(i - nb, 0), 0)),
        ],
        out_shape=[
            jax.ShapeDtypeStruct((M, 2), jnp.float32),
            jax.ShapeDtypeStruct((M, 2), jnp.int32),
        ],
        scratch_shapes=[
            pltpu.VMEM((M, E), jnp.float32),
            pltpu.VMEM((1, E), jnp.float32),
            pltpu.VMEM((1, E), jnp.float32),
        ],
    )(xf, W)
    return (scores, idx)
